# baseline (speedup 1.0000x reference)
import jax
import jax.numpy as jnp
from jax.experimental import pallas as pl
from jax.experimental.pallas import tpu as pltpu

_N = 10000
_BM = 200
_NBUF = 5


def _fused_kernel(adj_ref, x_ref, w1_ref, b1_ref, w2_ref, b2_ref,
                  out_ref, abuf, xb_ref, s2f_ref, s2b_ref, sems):
    nb = _N // _BM
    total = 2 * nb

    splits = ((0, 96), (96, 104))

    def chunk_copies(t):
        row = (t % nb) * _BM
        slot = jax.lax.rem(t, _NBUF)
        return [
            pltpu.make_async_copy(
                adj_ref.at[pl.ds(row + off, sz), :],
                abuf.at[slot, pl.ds(off, sz), :],
                sems.at[slot, k],
            )
            for k, (off, sz) in enumerate(splits)
        ]

    def start_chunk(t):
        for c in chunk_copies(t):
            c.start()

    for t in range(_NBUF):
        start_chunk(t)

    xb_ref[...] = x_ref[...].astype(jnp.bfloat16)

    def load_chunk(t):
        for c in chunk_copies(t):
            c.wait()
        return abuf[jax.lax.rem(t, _NBUF)].astype(jnp.bfloat16)

    def phase0_body(t, carry):
        a = load_chunk(t)
        s = jnp.dot(a, xb_ref[...], preferred_element_type=jnp.float32)
        h = jnp.dot(s.astype(jnp.bfloat16), w1_ref[...].astype(jnp.bfloat16),
                    preferred_element_type=jnp.float32)
        h = jnp.maximum(h + b1_ref[...], 0.0)
        s2 = jnp.dot(h.astype(jnp.bfloat16), w2_ref[...].astype(jnp.bfloat16),
                     preferred_element_type=jnp.float32)
        s2f_ref[pl.ds(t * _BM, _BM), :] = s2
        start_chunk(t + _NBUF)
        return carry

    def phase1_body(t, carry):
        a = load_chunk(t)
        o = jnp.dot(a, s2b_ref[...], preferred_element_type=jnp.float32)
        out_ref[pl.ds((t - nb) * _BM, _BM), :] = \
            jnp.maximum(o + b2_ref[...], 0.0)

        @pl.when(t + _NBUF < total)
        def _():
            start_chunk(t + _NBUF)
        return carry

    jax.lax.fori_loop(0, nb, phase0_body, 0, unroll=False)
    s2b_ref[...] = s2f_ref[...].astype(jnp.bfloat16)
    jax.lax.fori_loop(nb, total, phase1_body, 0, unroll=False)


def kernel(x, adj, W1, b1, W2, b2):
    n, f_in = x.shape
    h1 = W1.shape[1]
    h2 = W2.shape[1]

    b1_2d = b1.reshape(1, h1)
    b2_2d = b2.reshape(1, h2)

    vmem = pl.BlockSpec(memory_space=pltpu.MemorySpace.VMEM)
    out = pl.pallas_call(
        _fused_kernel,
        in_specs=[
            pl.BlockSpec(memory_space=pl.ANY),
            vmem, vmem, vmem, vmem, vmem,
        ],
        out_specs=vmem,
        out_shape=jax.ShapeDtypeStruct((n, h2), jnp.float32),
        scratch_shapes=[
            pltpu.VMEM((_NBUF, _BM, _N), jnp.float32),
            pltpu.VMEM((n, f_in), jnp.bfloat16),
            pltpu.VMEM((_N, h2), jnp.float32),
            pltpu.VMEM((_N, h2), jnp.bfloat16),
            pltpu.SemaphoreType.DMA((_NBUF, 2)),
        ],
    )(adj, x, W1, b1_2d, W2, b2_2d)

    return out
